# baseline (speedup 1.0000x reference)
import jax
import jax.numpy as jnp
from jax import lax
from jax.experimental import pallas as pl
from jax.experimental.pallas import tpu as pltpu
from jax.experimental.pallas import tpu_sc as plsc

_GDN = lax.GatherDimensionNumbers(
    offset_dims=(), collapsed_slice_dims=(0,), start_index_map=(0,))


def _lane_shuffle(v, idx16):
    return lax.gather(v, idx16[:, None], _GDN, (1,),
                      mode=lax.GatherScatterMode.PROMISE_IN_BOUNDS)


N = 10000
E = 320000
DF = 128
DE = 16
H = 8
C = 16
HC = H * C
NC = 40
NEG = 0.2

NPAD = 10240
HH = HC // 2
PW = 80
B = 128
CHUNKS = 160
SUPERS = CHUNKS // 4
EPT = CHUNKS * B
EPAD = EPT * 16
ROWS_PER_TILE = NPAD // 16
ZROWS = 128


def _lin2_body(h_ref, wl_ref, bl_ref, wr_ref, br_ref, xl_ref, xr_ref):
    hb = h_ref[...]
    xl = jnp.dot(hb, wl_ref[...], preferred_element_type=jnp.float32) + bl_ref[...]
    xr = jnp.dot(hb, wr_ref[...], preferred_element_type=jnp.float32) + br_ref[...]
    xl_ref[0] = xl[:, :HH]
    xl_ref[1] = xl[:, HH:]
    xr_ref[0] = xr[:, :HH]
    xr_ref[1] = xr[:, HH:]


def _lin2(h, wl, bl, wr, br):
    blk = 2048
    grid = (NPAD // blk,)
    return pl.pallas_call(
        _lin2_body,
        grid=grid,
        in_specs=[
            pl.BlockSpec((blk, DF), lambda i: (i, 0)),
            pl.BlockSpec((DF, HC), lambda i: (0, 0)),
            pl.BlockSpec((1, HC), lambda i: (0, 0)),
            pl.BlockSpec((DF, HC), lambda i: (0, 0)),
            pl.BlockSpec((1, HC), lambda i: (0, 0)),
        ],
        out_specs=[
            pl.BlockSpec((2, blk, HH), lambda i: (0, i, 0)),
            pl.BlockSpec((2, blk, HH), lambda i: (0, i, 0)),
        ],
        out_shape=[
            jax.ShapeDtypeStruct((2, NPAD, HH), jnp.float32),
            jax.ShapeDtypeStruct((2, NPAD, HH), jnp.float32),
        ],
    )(h, wl, bl, wr, br)


def _ea_body(ea_ref, w0_ref, w1_ref, w2_ref, o0_ref, o1_ref, o2_ref):
    a = ea_ref[...]
    for w_ref, o_ref in ((w0_ref, o0_ref), (w1_ref, o1_ref), (w2_ref, o2_ref)):
        o = jnp.dot(a, w_ref[...], preferred_element_type=jnp.float32)
        o_ref[0] = o[:, :HH]
        o_ref[1] = o[:, HH:]


def _ea_all(ea_pad, we0, we1, we2):
    blk = 4096
    grid = (EPAD // blk,)
    return pl.pallas_call(
        _ea_body,
        grid=grid,
        in_specs=[
            pl.BlockSpec((blk, DE), lambda i: (i, 0)),
            pl.BlockSpec((DE, HC), lambda i: (0, 0)),
            pl.BlockSpec((DE, HC), lambda i: (0, 0)),
            pl.BlockSpec((DE, HC), lambda i: (0, 0)),
        ],
        out_specs=[pl.BlockSpec((2, blk, HH), lambda i: (0, i, 0))] * 3,
        out_shape=[jax.ShapeDtypeStruct((2, EPAD, HH), jnp.float32)] * 3,
    )(ea_pad, we0, we1, we2)


def _norm_h(acc_ref, b):
    r16 = lax.broadcasted_iota(jnp.int32, (16, HH), 0)
    c64 = lax.broadcasted_iota(jnp.int32, (16, HH), 1)
    k = jnp.where(c64 // C == r16 // 4, 0.25, 0.0)
    a0 = acc_ref[0]
    a1 = acc_ref[1]
    d0 = jnp.dot(a0[:, HH:], k, preferred_element_type=jnp.float32,
                 precision=lax.Precision.HIGHEST)
    d1 = jnp.dot(a1[:, HH:], k, preferred_element_type=jnp.float32,
                 precision=lax.Precision.HIGHEST)
    h0 = a0[:, :HH] / (d0 + 1e-16)
    h1 = a1[:, :HH] / (d1 + 1e-16)
    hv = jnp.concatenate([h0, h1], axis=1) + b
    return jnp.maximum(hv, NEG * hv)


def _post_body(acc_ref, b_ref, h_ref):
    h_ref[...] = _norm_h(acc_ref, b_ref[...])


def _post(acc, b):
    blk = 2048
    grid = (NPAD // blk,)
    return pl.pallas_call(
        _post_body,
        grid=grid,
        in_specs=[
            pl.BlockSpec((2, blk, PW), lambda i: (0, i, 0)),
            pl.BlockSpec((1, HC), lambda i: (0, 0)),
        ],
        out_specs=pl.BlockSpec((blk, HC), lambda i: (i, 0)),
        out_shape=jax.ShapeDtypeStruct((NPAD, HC), jnp.float32),
    )(acc, b)


def _final_body(acc_ref, b_ref, wlin_ref, blin_ref, o_ref):
    h = _norm_h(acc_ref, b_ref[...])
    o_ref[...] = jnp.dot(h, wlin_ref[...], preferred_element_type=jnp.float32) + blin_ref[...]


def _final(acc, b, wlin_pad, blin_pad):
    blk = 2048
    grid = (NPAD // blk,)
    return pl.pallas_call(
        _final_body,
        grid=grid,
        in_specs=[
            pl.BlockSpec((2, blk, PW), lambda i: (0, i, 0)),
            pl.BlockSpec((1, HC), lambda i: (0, 0)),
            pl.BlockSpec((HC, HC), lambda i: (0, 0)),
            pl.BlockSpec((1, HC), lambda i: (0, 0)),
        ],
        out_specs=pl.BlockSpec((blk, HC), lambda i: (i, 0)),
        out_shape=jax.ShapeDtypeStruct((NPAD, HC), jnp.float32),
    )(acc, b, wlin_pad, blin_pad)


def _sc_edge_body(xl_hbm, xr_hbm, ea_hbm, sd_hbm, att_hbm, out_hbm,
                  idx0, idx1, idx2, idx3,
                  idg0, idg1, idg2, idg3,
                  xlb0, xlb1, xrb0, xrb1, eab0, eab1,
                  pay, attb, shared,
                  i0, i1, i2, i3, gl0, gl1, gr0, gr1, ge0, ge1):
    c = lax.axis_index("c")
    s = lax.axis_index("s")
    lane = lax.iota(jnp.int32, 16)
    lane4 = lax.shift_right_logical(lane, 2)
    zeros16 = jnp.zeros((16,), jnp.float32)

    idxs = (idx0, idx1, idx2, idx3)
    idgs = (idg0, idg1, idg2, idg3)
    isems = (i0, i1, i2, i3)
    xlbs = (xlb0, xlb1)
    xrbs = (xrb0, xrb1)
    eabs = (eab0, eab1)
    glsems = (gl0, gl1)
    grsems = (gr0, gr1)
    gesems = (ge0, ge1)

    pltpu.sync_copy(att_hbm, attb)

    def zrow(i, _):
        for j in range(PW // 16):
            pay[i, pl.ds(j * 16, 16)] = zeros16
        return 0
    lax.fori_loop(0, B, zrow, 0)

    rowbase = s * ROWS_PER_TILE

    def zshared(j, _):
        pltpu.sync_copy(pay.at[pl.ds(0, ZROWS)],
                        shared.at[pl.ds(rowbase + j * ZROWS, ZROWS)])
        return 0
    lax.fori_loop(0, ROWS_PER_TILE // ZROWS, zshared, 0)
    plsc.subcore_barrier()

    ebase = s * EPT

    def start_idx(m, slot):
        pltpu.async_copy(sd_hbm.at[:, pl.ds(ebase + m * B, B)],
                         idxs[slot], isems[slot])

    def wait_idx(slot):
        pltpu.make_async_copy(sd_hbm.at[:, pl.ds(ebase, B)],
                              idxs[slot], isems[slot]).wait()

    def offset_idx(slot):
        del slot

    def start_xl(slot, pb):
        pltpu.async_copy(xl_hbm.at[c].at[idxs[slot].at[0]], xlbs[pb],
                         glsems[pb])

    def wait_xl(pb):
        pltpu.make_async_copy(xl_hbm.at[c].at[idxs[0].at[0]], xlbs[pb],
                              glsems[pb]).wait()

    def start_xr(slot, pb):
        pltpu.async_copy(xr_hbm.at[c].at[idxs[slot].at[1]], xrbs[pb],
                         grsems[pb])

    def wait_xr(pb):
        pltpu.make_async_copy(xr_hbm.at[c].at[idxs[0].at[1]], xrbs[pb],
                              grsems[pb]).wait()

    def start_ea(m, pb):
        pltpu.async_copy(ea_hbm.at[c, pl.ds(ebase + m * B, B)], eabs[pb],
                         gesems[pb])

    def wait_ea(pb):
        pltpu.make_async_copy(ea_hbm.at[c, pl.ds(ebase, B)], eabs[pb],
                              gesems[pb]).wait()

    def compute_chunk(pb, slot):
        xlb, xrb, eab = xlbs[pb], xrbs[pb], eabs[pb]

        @plsc.parallel_loop(0, B, step=1, unroll=8)
        def edge(e):
            exm = zeros16
            for h in range(H // 2):
                xlv = xlb[e, pl.ds(h * 16, 16)]
                sv = (xlv + xrb[e, pl.ds(h * 16, 16)]
                      + eab[e, pl.ds(h * 16, 16)])
                ev = jnp.maximum(sv, sv * NEG)
                t = ev * attb[pl.ds(c * HH + h * 16, 16)]
                for sh in (8, 4, 2, 1):
                    t = t + _lane_shuffle(t, lane ^ sh)
                exs = jnp.exp(t)
                pay[e, pl.ds(h * 16, 16)] = xlv * exs
                exm = jnp.where(lane4 == h, exs, exm)
            pay[e, pl.ds(HH, 16)] = exm
        pltpu.sync_copy(pay, shared.at[idxs[slot].at[1]], add=True)

    start_idx(0, 0)
    start_idx(1, 1)

    last = CHUNKS - 1

    def pair(k0, b2):
        s0 = 2 * b2
        s1 = 2 * b2 + 1
        wait_idx(s0)
        offset_idx(s0)
        start_xl(s0, 0)
        start_xr(s0, 0)
        start_ea(k0, 0)
        wait_idx(s1)
        offset_idx(s1)
        wait_xl(0)
        wait_xr(0)
        wait_ea(0)
        start_xl(s1, 1)
        start_xr(s1, 1)
        start_ea(k0 + 1, 1)
        compute_chunk(0, s0)
        wait_xl(1)
        wait_xr(1)
        wait_ea(1)
        compute_chunk(1, s1)
        start_idx(jnp.minimum(k0 + 2, last), s0 ^ 2)
        start_idx(jnp.minimum(k0 + 3, last), s1 ^ 2)

    def superstep(j, _):
        for b2 in range(2):
            pair(4 * j + 2 * b2, b2)
        return 0
    lax.fori_loop(0, SUPERS, superstep, 0)

    wait_idx(0)
    wait_idx(1)

    plsc.subcore_barrier()

    def cpout(j, _):
        r = rowbase + j * ZROWS
        pltpu.sync_copy(shared.at[pl.ds(r, ZROWS)],
                        out_hbm.at[c, pl.ds(r, ZROWS)])
        return 0
    lax.fori_loop(0, ROWS_PER_TILE // ZROWS, cpout, 0)


def _sc_edge(xl, xr, ea, sd, att_flat):
    fn = pl.kernel(
        _sc_edge_body,
        out_type=jax.ShapeDtypeStruct((2, NPAD, PW), jnp.float32),
        mesh=plsc.VectorSubcoreMesh(core_axis_name="c", subcore_axis_name="s"),
        compiler_params=pltpu.CompilerParams(use_tc_tiling_on_sc=False),
        scratch_types=[pltpu.VMEM((2, B), jnp.int32)] * 8
        + [pltpu.VMEM((B, HH), jnp.float32)] * 6 + [
            pltpu.VMEM((B, PW), jnp.float32),
            pltpu.VMEM((HC,), jnp.float32),
            pltpu.VMEM_SHARED((NPAD, PW), jnp.float32),
        ] + [pltpu.SemaphoreType.DMA] * 10,
    )
    return fn(xl, xr, ea, sd, att_flat)


def kernel(x, edge_index, edge_attr, Wl0, bl0, Wr0, br0, We0, att0, b0,
           Wl1, bl1, Wr1, br1, We1, att1, b1, Wl2, bl2, Wr2, br2, We2, att2,
           b2, Wlin, blin):
    sd = edge_index.astype(jnp.int32)
    sd_p = jnp.pad(sd, ((0, 0), (0, EPAD - E)), constant_values=NPAD - 1)
    ea_pad = jnp.pad(edge_attr, ((0, EPAD - E), (0, 0)))
    x_p = jnp.pad(x, ((0, NPAD - N), (0, 0)))

    eas = _ea_all(ea_pad, We0, We1, We2)
    layers = (
        (Wl0, bl0, Wr0, br0, att0, b0),
        (Wl1, bl1, Wr1, br1, att1, b1),
        (Wl2, bl2, Wr2, br2, att2, b2),
    )

    h = x_p
    for l, (wl, bl, wr, br, att, b) in enumerate(layers):
        xl, xr = _lin2(h, wl, bl.reshape(1, HC), wr, br.reshape(1, HC))
        acc = _sc_edge(xl, xr, eas[l], sd_p, att.reshape(HC))
        if l < 2:
            h = _post(acc, b.reshape(1, HC))
        else:
            wlin_pad = jnp.pad(Wlin, ((0, 0), (0, HC - NC)))
            blin_pad = jnp.pad(blin, (0, HC - NC)).reshape(1, HC)
            out = _final(acc, b.reshape(1, HC), wlin_pad, blin_pad)
    return out[:N, :NC]


# speedup vs baseline: 1.1643x; 1.1643x over previous
import jax
import jax.numpy as jnp
from jax import lax
from jax.experimental import pallas as pl
from jax.experimental.pallas import tpu as pltpu
from jax.experimental.pallas import tpu_sc as plsc

_GDN = lax.GatherDimensionNumbers(
    offset_dims=(), collapsed_slice_dims=(0,), start_index_map=(0,))


def _lane_shuffle(v, idx16):
    return lax.gather(v, idx16[:, None], _GDN, (1,),
                      mode=lax.GatherScatterMode.PROMISE_IN_BOUNDS)


N = 10000
E = 320000
DF = 128
DE = 16
H = 8
C = 16
HC = H * C
NC = 40
NEG = 0.2

NPAD = 10240
HH = HC // 2
PW = 80
B = 128
CHUNKS = 160
SUPERS = CHUNKS // 4
EPT = CHUNKS * B
EPAD = EPT * 16
ROWS_PER_TILE = NPAD // 16
ZROWS = 128


def _lin2_body(h_ref, wl_ref, bl_ref, wr_ref, br_ref, xl_ref, xr_ref):
    hb = h_ref[...]
    xl = jnp.dot(hb, wl_ref[...], preferred_element_type=jnp.float32) + bl_ref[...]
    xr = jnp.dot(hb, wr_ref[...], preferred_element_type=jnp.float32) + br_ref[...]
    xl_ref[0] = xl[:, :HH]
    xl_ref[1] = xl[:, HH:]
    xr_ref[0] = xr[:, :HH]
    xr_ref[1] = xr[:, HH:]


def _lin2(h, wl, bl, wr, br):
    blk = 2048
    grid = (NPAD // blk,)
    return pl.pallas_call(
        _lin2_body,
        grid=grid,
        in_specs=[
            pl.BlockSpec((blk, DF), lambda i: (i, 0)),
            pl.BlockSpec((DF, HC), lambda i: (0, 0)),
            pl.BlockSpec((1, HC), lambda i: (0, 0)),
            pl.BlockSpec((DF, HC), lambda i: (0, 0)),
            pl.BlockSpec((1, HC), lambda i: (0, 0)),
        ],
        out_specs=[
            pl.BlockSpec((2, blk, HH), lambda i: (0, i, 0)),
            pl.BlockSpec((2, blk, HH), lambda i: (0, i, 0)),
        ],
        out_shape=[
            jax.ShapeDtypeStruct((2, NPAD, HH), jnp.float32),
            jax.ShapeDtypeStruct((2, NPAD, HH), jnp.float32),
        ],
    )(h, wl, bl, wr, br)


def _ea_body(ea_ref, w0_ref, w1_ref, w2_ref, o0_ref, o1_ref, o2_ref):
    a = ea_ref[...]
    for w_ref, o_ref in ((w0_ref, o0_ref), (w1_ref, o1_ref), (w2_ref, o2_ref)):
        o = jnp.dot(a, w_ref[...], preferred_element_type=jnp.float32)
        o_ref[0] = o[:, :HH]
        o_ref[1] = o[:, HH:]


def _ea_all(ea_pad, we0, we1, we2):
    blk = 4096
    grid = (EPAD // blk,)
    return pl.pallas_call(
        _ea_body,
        grid=grid,
        in_specs=[
            pl.BlockSpec((blk, DE), lambda i: (i, 0)),
            pl.BlockSpec((DE, HC), lambda i: (0, 0)),
            pl.BlockSpec((DE, HC), lambda i: (0, 0)),
            pl.BlockSpec((DE, HC), lambda i: (0, 0)),
        ],
        out_specs=[pl.BlockSpec((2, blk, HH), lambda i: (0, i, 0))] * 3,
        out_shape=[jax.ShapeDtypeStruct((2, EPAD, HH), jnp.float32)] * 3,
    )(ea_pad, we0, we1, we2)


def _norm_h(acc_ref, b):
    r16 = lax.broadcasted_iota(jnp.int32, (16, HH), 0)
    c64 = lax.broadcasted_iota(jnp.int32, (16, HH), 1)
    k = jnp.where(c64 // C == r16 // 4, 0.25, 0.0)
    a0 = acc_ref[0]
    a1 = acc_ref[1]
    d0 = jnp.dot(a0[:, HH:], k, preferred_element_type=jnp.float32,
                 precision=lax.Precision.HIGHEST)
    d1 = jnp.dot(a1[:, HH:], k, preferred_element_type=jnp.float32,
                 precision=lax.Precision.HIGHEST)
    h0 = a0[:, :HH] / (d0 + 1e-16)
    h1 = a1[:, :HH] / (d1 + 1e-16)
    hv = jnp.concatenate([h0, h1], axis=1) + b
    return jnp.maximum(hv, NEG * hv)


def _post_body(acc_ref, b_ref, h_ref):
    h_ref[...] = _norm_h(acc_ref, b_ref[...])


def _post(acc, b):
    blk = 2048
    grid = (NPAD // blk,)
    return pl.pallas_call(
        _post_body,
        grid=grid,
        in_specs=[
            pl.BlockSpec((2, blk, PW), lambda i: (0, i, 0)),
            pl.BlockSpec((1, HC), lambda i: (0, 0)),
        ],
        out_specs=pl.BlockSpec((blk, HC), lambda i: (i, 0)),
        out_shape=jax.ShapeDtypeStruct((NPAD, HC), jnp.float32),
    )(acc, b)


def _final_body(acc_ref, b_ref, wlin_ref, blin_ref, o_ref):
    h = _norm_h(acc_ref, b_ref[...])
    o_ref[...] = jnp.dot(h, wlin_ref[...], preferred_element_type=jnp.float32) + blin_ref[...]


def _final(acc, b, wlin_pad, blin_pad):
    blk = 2048
    grid = (NPAD // blk,)
    return pl.pallas_call(
        _final_body,
        grid=grid,
        in_specs=[
            pl.BlockSpec((2, blk, PW), lambda i: (0, i, 0)),
            pl.BlockSpec((1, HC), lambda i: (0, 0)),
            pl.BlockSpec((HC, HC), lambda i: (0, 0)),
            pl.BlockSpec((1, HC), lambda i: (0, 0)),
        ],
        out_specs=pl.BlockSpec((blk, HC), lambda i: (i, 0)),
        out_shape=jax.ShapeDtypeStruct((NPAD, HC), jnp.float32),
    )(acc, b, wlin_pad, blin_pad)


def _sc_edge_body(xl_hbm, xr_hbm, ea_hbm, sd_hbm, att_hbm, out_hbm,
                  idx0, idx1, idx2, idx3,
                  xlb0, xlb1, xrb0, xrb1, eab0, eab1,
                  pay0, pay1, attb, shared,
                  i0, i1, i2, i3, gl0, gl1, gr0, gr1, ge0, ge1,
                  ss0, ss1):
    c = lax.axis_index("c")
    s = lax.axis_index("s")
    lane = lax.iota(jnp.int32, 16)
    lane4 = lax.shift_right_logical(lane, 2)
    zeros16 = jnp.zeros((16,), jnp.float32)

    idxs = (idx0, idx1, idx2, idx3)
    isems = (i0, i1, i2, i3)
    xlbs = (xlb0, xlb1)
    xrbs = (xrb0, xrb1)
    eabs = (eab0, eab1)
    pays = (pay0, pay1)
    glsems = (gl0, gl1)
    grsems = (gr0, gr1)
    gesems = (ge0, ge1)
    ssems = (ss0, ss1)

    pltpu.sync_copy(att_hbm, attb)

    def zrow(i, _):
        for j in range(PW // 16):
            pay0[i, pl.ds(j * 16, 16)] = zeros16
        return 0
    lax.fori_loop(0, B, zrow, 0)

    rowbase = s * ROWS_PER_TILE

    def zshared(j, _):
        pltpu.sync_copy(pay0.at[pl.ds(0, ZROWS)],
                        shared.at[pl.ds(rowbase + j * ZROWS, ZROWS)])
        return 0
    lax.fori_loop(0, ROWS_PER_TILE // ZROWS, zshared, 0)
    plsc.subcore_barrier()

    ebase = s * EPT

    def start_idx(m, slot):
        pltpu.async_copy(sd_hbm.at[:, pl.ds(ebase + m * B, B)],
                         idxs[slot], isems[slot])

    def wait_idx(slot):
        pltpu.make_async_copy(sd_hbm.at[:, pl.ds(ebase, B)],
                              idxs[slot], isems[slot]).wait()

    def start_xl(slot, pb):
        pltpu.async_copy(xl_hbm.at[c].at[idxs[slot].at[0]], xlbs[pb],
                         glsems[pb])

    def wait_xl(pb):
        pltpu.make_async_copy(xl_hbm.at[c].at[idxs[0].at[0]], xlbs[pb],
                              glsems[pb]).wait()

    def start_xr(slot, pb):
        pltpu.async_copy(xr_hbm.at[c].at[idxs[slot].at[1]], xrbs[pb],
                         grsems[pb])

    def wait_xr(pb):
        pltpu.make_async_copy(xr_hbm.at[c].at[idxs[0].at[1]], xrbs[pb],
                              grsems[pb]).wait()

    def start_ea(m, pb):
        pltpu.async_copy(ea_hbm.at[c, pl.ds(ebase + m * B, B)], eabs[pb],
                         gesems[pb])

    def wait_ea(pb):
        pltpu.make_async_copy(ea_hbm.at[c, pl.ds(ebase, B)], eabs[pb],
                              gesems[pb]).wait()

    def wait_scatter(pb):
        pltpu.make_async_copy(pays[pb], shared.at[idxs[0].at[1]],
                              ssems[pb]).wait()

    def compute_chunk(pb, slot):
        xlb, xrb, eab, pay = xlbs[pb], xrbs[pb], eabs[pb], pays[pb]

        @plsc.parallel_loop(0, B, step=1, unroll=4)
        def edge(e):
            exm = zeros16
            for h in range(H // 2):
                xlv = xlb[e, pl.ds(h * 16, 16)]
                sv = (xlv + xrb[e, pl.ds(h * 16, 16)]
                      + eab[e, pl.ds(h * 16, 16)])
                ev = jnp.maximum(sv, sv * NEG)
                t = ev * attb[pl.ds(c * HH + h * 16, 16)]
                for sh in (8, 4, 2, 1):
                    t = t + _lane_shuffle(t, lane ^ sh)
                exs = jnp.exp(t)
                pay[e, pl.ds(h * 16, 16)] = xlv * exs
                exm = jnp.where(lane4 == h, exs, exm)
            pay[e, pl.ds(HH, 16)] = exm
        pltpu.async_copy(pay, shared.at[idxs[slot].at[1]], ssems[pb],
                         add=True)

    def start_g(m, slot, pb):
        start_xl(slot, pb)
        start_xr(slot, pb)
        start_ea(m, pb)

    def wait_g(pb):
        wait_xl(pb)
        wait_xr(pb)
        wait_ea(pb)

    for slot in range(4):
        start_idx(slot, slot)

    last = CHUNKS - 1

    def quad(j, _):
        k0 = 4 * j
        wait_idx(0)
        start_g(k0, 0, 0)
        wait_g(0)
        wait_idx(1)
        start_g(k0 + 1, 1, 1)
        compute_chunk(0, 0)
        wait_g(1)
        wait_idx(2)
        start_g(k0 + 2, 2, 0)
        compute_chunk(1, 1)
        wait_g(0)
        wait_idx(3)
        start_g(k0 + 3, 3, 1)
        wait_scatter(0)
        compute_chunk(0, 2)
        wait_g(1)
        wait_scatter(1)
        compute_chunk(1, 3)
        wait_scatter(0)
        wait_scatter(1)
        for i in range(4):
            start_idx(jnp.minimum(k0 + 4 + i, last), i)
        return 0
    lax.fori_loop(0, SUPERS, quad, 0)

    for slot in range(4):
        wait_idx(slot)

    plsc.subcore_barrier()

    def cpout(j, _):
        r = rowbase + j * ZROWS
        pltpu.sync_copy(shared.at[pl.ds(r, ZROWS)],
                        out_hbm.at[c, pl.ds(r, ZROWS)])
        return 0
    lax.fori_loop(0, ROWS_PER_TILE // ZROWS, cpout, 0)


def _sc_edge(xl, xr, ea, sd, att_flat):
    fn = pl.kernel(
        _sc_edge_body,
        out_type=jax.ShapeDtypeStruct((2, NPAD, PW), jnp.float32),
        mesh=plsc.VectorSubcoreMesh(core_axis_name="c", subcore_axis_name="s"),
        compiler_params=pltpu.CompilerParams(use_tc_tiling_on_sc=False),
        scratch_types=[pltpu.VMEM((2, B), jnp.int32)] * 4
        + [pltpu.VMEM((B, HH), jnp.float32)] * 6 + [
            pltpu.VMEM((B, PW), jnp.float32),
            pltpu.VMEM((B, PW), jnp.float32),
            pltpu.VMEM((HC,), jnp.float32),
            pltpu.VMEM_SHARED((NPAD, PW), jnp.float32),
        ] + [pltpu.SemaphoreType.DMA] * 12,
    )
    return fn(xl, xr, ea, sd, att_flat)


def kernel(x, edge_index, edge_attr, Wl0, bl0, Wr0, br0, We0, att0, b0,
           Wl1, bl1, Wr1, br1, We1, att1, b1, Wl2, bl2, Wr2, br2, We2, att2,
           b2, Wlin, blin):
    sd = edge_index.astype(jnp.int32)
    sd_p = jnp.pad(sd, ((0, 0), (0, EPAD - E)), constant_values=NPAD - 1)
    ea_pad = jnp.pad(edge_attr, ((0, EPAD - E), (0, 0)))
    x_p = jnp.pad(x, ((0, NPAD - N), (0, 0)))

    eas = _ea_all(ea_pad, We0, We1, We2)
    layers = (
        (Wl0, bl0, Wr0, br0, att0, b0),
        (Wl1, bl1, Wr1, br1, att1, b1),
        (Wl2, bl2, Wr2, br2, att2, b2),
    )

    h = x_p
    for l, (wl, bl, wr, br, att, b) in enumerate(layers):
        xl, xr = _lin2(h, wl, bl.reshape(1, HC), wr, br.reshape(1, HC))
        acc = _sc_edge(xl, xr, eas[l], sd_p, att.reshape(HC))
        if l < 2:
            h = _post(acc, b.reshape(1, HC))
        else:
            wlin_pad = jnp.pad(Wlin, ((0, 0), (0, HC - NC)))
            blin_pad = jnp.pad(blin, (0, HC - NC)).reshape(1, HC)
            out = _final(acc, b.reshape(1, HC), wlin_pad, blin_pad)
    return out[:N, :NC]


# speedup vs baseline: 1.1686x; 1.0037x over previous
import jax
import jax.numpy as jnp
from jax import lax
from jax.experimental import pallas as pl
from jax.experimental.pallas import tpu as pltpu
from jax.experimental.pallas import tpu_sc as plsc

_GDN = lax.GatherDimensionNumbers(
    offset_dims=(), collapsed_slice_dims=(0,), start_index_map=(0,))


def _lane_shuffle(v, idx16):
    return lax.gather(v, idx16[:, None], _GDN, (1,),
                      mode=lax.GatherScatterMode.PROMISE_IN_BOUNDS)


N = 10000
E = 320000
DF = 128
DE = 16
H = 8
C = 16
HC = H * C
NC = 40
NEG = 0.2

NPAD = 10240
HH = HC // 2
PW = 80
B = 128
CHUNKS = 160
SUPERS = CHUNKS // 4
EPT = CHUNKS * B
EPAD = EPT * 16
ROWS_PER_TILE = NPAD // 16
ZROWS = 128


def _lin2_body(h_ref, wl_ref, bl_ref, wr_ref, br_ref, xl_ref, xr_ref):
    hb = h_ref[...]
    xl = jnp.dot(hb, wl_ref[...], preferred_element_type=jnp.float32) + bl_ref[...]
    xr = jnp.dot(hb, wr_ref[...], preferred_element_type=jnp.float32) + br_ref[...]
    xl_ref[0] = xl[:, :HH]
    xl_ref[1] = xl[:, HH:]
    xr_ref[0] = xr[:, :HH]
    xr_ref[1] = xr[:, HH:]


def _lin2(h, wl, bl, wr, br):
    blk = 2048
    grid = (NPAD // blk,)
    return pl.pallas_call(
        _lin2_body,
        grid=grid,
        in_specs=[
            pl.BlockSpec((blk, DF), lambda i: (i, 0)),
            pl.BlockSpec((DF, HC), lambda i: (0, 0)),
            pl.BlockSpec((1, HC), lambda i: (0, 0)),
            pl.BlockSpec((DF, HC), lambda i: (0, 0)),
            pl.BlockSpec((1, HC), lambda i: (0, 0)),
        ],
        out_specs=[
            pl.BlockSpec((2, blk, HH), lambda i: (0, i, 0)),
            pl.BlockSpec((2, blk, HH), lambda i: (0, i, 0)),
        ],
        out_shape=[
            jax.ShapeDtypeStruct((2, NPAD, HH), jnp.float32),
            jax.ShapeDtypeStruct((2, NPAD, HH), jnp.float32),
        ],
    )(h, wl, bl, wr, br)


def _ea_body(ea_ref, w0_ref, w1_ref, w2_ref, o0_ref, o1_ref, o2_ref):
    a = ea_ref[...]
    for w_ref, o_ref in ((w0_ref, o0_ref), (w1_ref, o1_ref), (w2_ref, o2_ref)):
        o = jnp.dot(a, w_ref[...], preferred_element_type=jnp.float32)
        o_ref[0] = o[:, :HH]
        o_ref[1] = o[:, HH:]


def _ea_all(ea_pad, we0, we1, we2):
    blk = 4096
    grid = (EPAD // blk,)
    return pl.pallas_call(
        _ea_body,
        grid=grid,
        in_specs=[
            pl.BlockSpec((blk, DE), lambda i: (i, 0)),
            pl.BlockSpec((DE, HC), lambda i: (0, 0)),
            pl.BlockSpec((DE, HC), lambda i: (0, 0)),
            pl.BlockSpec((DE, HC), lambda i: (0, 0)),
        ],
        out_specs=[pl.BlockSpec((2, blk, HH), lambda i: (0, i, 0))] * 3,
        out_shape=[jax.ShapeDtypeStruct((2, EPAD, HH), jnp.float32)] * 3,
    )(ea_pad, we0, we1, we2)


def _norm_h(acc_ref, b):
    r16 = lax.broadcasted_iota(jnp.int32, (16, HH), 0)
    c64 = lax.broadcasted_iota(jnp.int32, (16, HH), 1)
    k = jnp.where(c64 // C == r16 // 4, 0.25, 0.0)
    a0 = acc_ref[0]
    a1 = acc_ref[1]
    d0 = jnp.dot(a0[:, HH:], k, preferred_element_type=jnp.float32,
                 precision=lax.Precision.HIGHEST)
    d1 = jnp.dot(a1[:, HH:], k, preferred_element_type=jnp.float32,
                 precision=lax.Precision.HIGHEST)
    h0 = a0[:, :HH] / (d0 + 1e-16)
    h1 = a1[:, :HH] / (d1 + 1e-16)
    hv = jnp.concatenate([h0, h1], axis=1) + b
    return jnp.maximum(hv, NEG * hv)


def _post_lin2_body(acc_ref, b_ref, wl_ref, bl_ref, wr_ref, br_ref,
                    xl_ref, xr_ref):
    hb = _norm_h(acc_ref, b_ref[...])
    xl = jnp.dot(hb, wl_ref[...], preferred_element_type=jnp.float32) + bl_ref[...]
    xr = jnp.dot(hb, wr_ref[...], preferred_element_type=jnp.float32) + br_ref[...]
    xl_ref[0] = xl[:, :HH]
    xl_ref[1] = xl[:, HH:]
    xr_ref[0] = xr[:, :HH]
    xr_ref[1] = xr[:, HH:]


def _post_lin2(acc, b, wl, bl, wr, br):
    blk = 2048
    grid = (NPAD // blk,)
    return pl.pallas_call(
        _post_lin2_body,
        grid=grid,
        in_specs=[
            pl.BlockSpec((2, blk, PW), lambda i: (0, i, 0)),
            pl.BlockSpec((1, HC), lambda i: (0, 0)),
            pl.BlockSpec((DF, HC), lambda i: (0, 0)),
            pl.BlockSpec((1, HC), lambda i: (0, 0)),
            pl.BlockSpec((DF, HC), lambda i: (0, 0)),
            pl.BlockSpec((1, HC), lambda i: (0, 0)),
        ],
        out_specs=[
            pl.BlockSpec((2, blk, HH), lambda i: (0, i, 0)),
            pl.BlockSpec((2, blk, HH), lambda i: (0, i, 0)),
        ],
        out_shape=[
            jax.ShapeDtypeStruct((2, NPAD, HH), jnp.float32),
            jax.ShapeDtypeStruct((2, NPAD, HH), jnp.float32),
        ],
    )(acc, b, wl, bl, wr, br)


def _final_body(acc_ref, b_ref, wlin_ref, blin_ref, o_ref):
    h = _norm_h(acc_ref, b_ref[...])
    o_ref[...] = jnp.dot(h, wlin_ref[...], preferred_element_type=jnp.float32) + blin_ref[...]


def _final(acc, b, wlin_pad, blin_pad):
    blk = 2048
    grid = (NPAD // blk,)
    return pl.pallas_call(
        _final_body,
        grid=grid,
        in_specs=[
            pl.BlockSpec((2, blk, PW), lambda i: (0, i, 0)),
            pl.BlockSpec((1, HC), lambda i: (0, 0)),
            pl.BlockSpec((HC, HC), lambda i: (0, 0)),
            pl.BlockSpec((1, HC), lambda i: (0, 0)),
        ],
        out_specs=pl.BlockSpec((blk, HC), lambda i: (i, 0)),
        out_shape=jax.ShapeDtypeStruct((NPAD, HC), jnp.float32),
    )(acc, b, wlin_pad, blin_pad)


def _sc_edge_body(xl_hbm, xr_hbm, ea_hbm, sd_hbm, att_hbm, out_hbm,
                  idx0, idx1, idx2, idx3,
                  xlb0, xlb1, xrb0, xrb1, eab0, eab1,
                  pay0, pay1, attb, shared,
                  i0, i1, i2, i3, gl0, gl1, gr0, gr1, ge0, ge1,
                  ss0, ss1):
    c = lax.axis_index("c")
    s = lax.axis_index("s")
    lane = lax.iota(jnp.int32, 16)
    lane4 = lax.shift_right_logical(lane, 2)
    zeros16 = jnp.zeros((16,), jnp.float32)

    idxs = (idx0, idx1, idx2, idx3)
    isems = (i0, i1, i2, i3)
    xlbs = (xlb0, xlb1)
    xrbs = (xrb0, xrb1)
    eabs = (eab0, eab1)
    pays = (pay0, pay1)
    glsems = (gl0, gl1)
    grsems = (gr0, gr1)
    gesems = (ge0, ge1)
    ssems = (ss0, ss1)

    pltpu.sync_copy(att_hbm, attb)

    def zrow(i, _):
        for j in range(PW // 16):
            pay0[i, pl.ds(j * 16, 16)] = zeros16
        return 0
    lax.fori_loop(0, B, zrow, 0)

    rowbase = s * ROWS_PER_TILE

    def zshared(j, _):
        pltpu.sync_copy(pay0.at[pl.ds(0, ZROWS)],
                        shared.at[pl.ds(rowbase + j * ZROWS, ZROWS)])
        return 0
    lax.fori_loop(0, ROWS_PER_TILE // ZROWS, zshared, 0)
    plsc.subcore_barrier()

    ebase = s * EPT

    def start_idx(m, slot):
        pltpu.async_copy(sd_hbm.at[:, pl.ds(ebase + m * B, B)],
                         idxs[slot], isems[slot])

    def wait_idx(slot):
        pltpu.make_async_copy(sd_hbm.at[:, pl.ds(ebase, B)],
                              idxs[slot], isems[slot]).wait()

    def start_xl(slot, pb):
        pltpu.async_copy(xl_hbm.at[c].at[idxs[slot].at[0]], xlbs[pb],
                         glsems[pb])

    def wait_xl(pb):
        pltpu.make_async_copy(xl_hbm.at[c].at[idxs[0].at[0]], xlbs[pb],
                              glsems[pb]).wait()

    def start_xr(slot, pb):
        pltpu.async_copy(xr_hbm.at[c].at[idxs[slot].at[1]], xrbs[pb],
                         grsems[pb])

    def wait_xr(pb):
        pltpu.make_async_copy(xr_hbm.at[c].at[idxs[0].at[1]], xrbs[pb],
                              grsems[pb]).wait()

    def start_ea(m, pb):
        pltpu.async_copy(ea_hbm.at[c, pl.ds(ebase + m * B, B)], eabs[pb],
                         gesems[pb])

    def wait_ea(pb):
        pltpu.make_async_copy(ea_hbm.at[c, pl.ds(ebase, B)], eabs[pb],
                              gesems[pb]).wait()

    def wait_scatter(pb):
        pltpu.make_async_copy(pays[pb], shared.at[idxs[0].at[1]],
                              ssems[pb]).wait()

    def compute_chunk(pb, slot):
        xlb, xrb, eab, pay = xlbs[pb], xrbs[pb], eabs[pb], pays[pb]

        @plsc.parallel_loop(0, B, step=1, unroll=4)
        def edge(e):
            exm = zeros16
            for h in range(H // 2):
                xlv = xlb[e, pl.ds(h * 16, 16)]
                sv = (xlv + xrb[e, pl.ds(h * 16, 16)]
                      + eab[e, pl.ds(h * 16, 16)])
                ev = jnp.maximum(sv, sv * NEG)
                t = ev * attb[pl.ds(c * HH + h * 16, 16)]
                for sh in (8, 4, 2, 1):
                    t = t + _lane_shuffle(t, lane ^ sh)
                exs = jnp.exp(t)
                pay[e, pl.ds(h * 16, 16)] = xlv * exs
                exm = jnp.where(lane4 == h, exs, exm)
            pay[e, pl.ds(HH, 16)] = exm
        pltpu.async_copy(pay, shared.at[idxs[slot].at[1]], ssems[pb],
                         add=True)

    def start_g(m, slot, pb):
        start_xl(slot, pb)
        start_xr(slot, pb)
        start_ea(m, pb)

    def wait_g(pb):
        wait_xl(pb)
        wait_xr(pb)
        wait_ea(pb)

    for slot in range(4):
        start_idx(slot, slot)

    last = CHUNKS - 1

    def quad(j, _):
        k0 = 4 * j
        wait_idx(0)
        start_g(k0, 0, 0)
        wait_g(0)
        wait_idx(1)
        start_g(k0 + 1, 1, 1)
        compute_chunk(0, 0)
        wait_g(1)
        wait_idx(2)
        start_g(k0 + 2, 2, 0)
        compute_chunk(1, 1)
        wait_g(0)
        wait_idx(3)
        start_g(k0 + 3, 3, 1)
        wait_scatter(0)
        compute_chunk(0, 2)
        wait_g(1)
        wait_scatter(1)
        compute_chunk(1, 3)
        wait_scatter(0)
        wait_scatter(1)
        for i in range(4):
            start_idx(jnp.minimum(k0 + 4 + i, last), i)
        return 0
    lax.fori_loop(0, SUPERS, quad, 0)

    for slot in range(4):
        wait_idx(slot)

    plsc.subcore_barrier()

    def cpout(j, _):
        r = rowbase + j * ZROWS
        pltpu.sync_copy(shared.at[pl.ds(r, ZROWS)],
                        out_hbm.at[c, pl.ds(r, ZROWS)])
        return 0
    lax.fori_loop(0, ROWS_PER_TILE // ZROWS, cpout, 0)


def _sc_edge(xl, xr, ea, sd, att_flat):
    fn = pl.kernel(
        _sc_edge_body,
        out_type=jax.ShapeDtypeStruct((2, NPAD, PW), jnp.float32),
        mesh=plsc.VectorSubcoreMesh(core_axis_name="c", subcore_axis_name="s"),
        compiler_params=pltpu.CompilerParams(use_tc_tiling_on_sc=False),
        scratch_types=[pltpu.VMEM((2, B), jnp.int32)] * 4
        + [pltpu.VMEM((B, HH), jnp.float32)] * 6 + [
            pltpu.VMEM((B, PW), jnp.float32),
            pltpu.VMEM((B, PW), jnp.float32),
            pltpu.VMEM((HC,), jnp.float32),
            pltpu.VMEM_SHARED((NPAD, PW), jnp.float32),
        ] + [pltpu.SemaphoreType.DMA] * 12,
    )
    return fn(xl, xr, ea, sd, att_flat)


def kernel(x, edge_index, edge_attr, Wl0, bl0, Wr0, br0, We0, att0, b0,
           Wl1, bl1, Wr1, br1, We1, att1, b1, Wl2, bl2, Wr2, br2, We2, att2,
           b2, Wlin, blin):
    sd = edge_index.astype(jnp.int32)
    sd_p = jnp.pad(sd, ((0, 0), (0, EPAD - E)), constant_values=NPAD - 1)
    ea_pad = jnp.pad(edge_attr, ((0, EPAD - E), (0, 0)))
    x_p = jnp.pad(x, ((0, NPAD - N), (0, 0)))

    eas = _ea_all(ea_pad, We0, We1, We2)
    layers = (
        (Wl0, bl0, Wr0, br0, att0, b0),
        (Wl1, bl1, Wr1, br1, att1, b1),
        (Wl2, bl2, Wr2, br2, att2, b2),
    )

    acc = None
    for l, (wl, bl, wr, br, att, b_prev) in enumerate(layers):
        if l == 0:
            xl, xr = _lin2(x_p, wl, bl.reshape(1, HC), wr, br.reshape(1, HC))
        else:
            xl, xr = _post_lin2(acc, layers[l - 1][5].reshape(1, HC),
                                wl, bl.reshape(1, HC), wr, br.reshape(1, HC))
        acc = _sc_edge(xl, xr, eas[l], sd_p, att.reshape(HC))
    wlin_pad = jnp.pad(Wlin, ((0, 0), (0, HC - NC)))
    blin_pad = jnp.pad(blin, (0, HC - NC)).reshape(1, HC)
    out = _final(acc, layers[2][5].reshape(1, HC), wlin_pad, blin_pad)
    return out[:N, :NC]
